# baseline (speedup 1.0000x reference)
import functools

import jax
import jax.numpy as jnp
from jax import lax
from jax.experimental import pallas as pl
from jax.experimental.pallas import tpu as pltpu
from jax.experimental.pallas import tpu_sc as plsc

B, C, H, W = 16, 3, 512, 512
HW = H * W
W2 = W // 2
NW = 32
PIX_PER_TILE = B * HW // NW
TROWS = B * HW // 2 + 8

P = 1024
NCHUNK = PIX_PER_TILE // P
NG = P // 16
NIDX = 2 * P
NDMA = NIDX // 128

IR = 8
IPIX = IR * W
IROWS = IPIX // 2
ICHUNK = 256 // IR

_MESH = plsc.VectorSubcoreMesh(core_axis_name="c", subcore_axis_name="s")
_CP = pltpu.CompilerParams(
    needs_layout_passes=False, use_tc_tiling_on_sc=False)


def _make_interleave():
    ibuf_set = [
        pltpu.VMEM((3, IR + 1, W), jnp.float32),
        pltpu.VMEM((IROWS, 16), jnp.float32),
    ]

    @functools.partial(
        pl.kernel,
        mesh=_MESH,
        out_type=jax.ShapeDtypeStruct((TROWS, 16), jnp.float32),
        compiler_params=_CP,
        scratch_types=ibuf_set + ibuf_set + [
            pltpu.SemaphoreType.DMA,
            pltpu.SemaphoreType.DMA,
            pltpu.SemaphoreType.DMA,
        ],
    )
    def interleave(img, table, *rest):
        bufs = (rest[0:2], rest[2:4])
        sem_i = (rest[4], rest[5])
        sem_t = rest[6]
        wid = lax.axis_index("s") * 2 + lax.axis_index("c")
        b = wid // 2
        half = wid % 2
        lane = lax.iota(jnp.int32, 16)

        def img_async(ch, par):
            iv = bufs[par][0]
            r0 = half * 256 + lax.rem(ch, ICHUNK) * IR
            re = jnp.minimum(r0 + IR, H - 1)
            pltpu.async_copy(img.at[pl.ds(3 * b, 3), pl.ds(r0, IR), :],
                             iv.at[:, pl.ds(0, IR), :], sem_i[par])
            pltpu.async_copy(img.at[pl.ds(3 * b, 3), pl.ds(re, 1), :],
                             iv.at[:, pl.ds(IR, 1), :], sem_i[par])

        def img_wait(par):
            iv = bufs[par][0]
            pltpu.make_async_copy(img.at[pl.ds(0, 3), pl.ds(0, IR), :],
                                  iv.at[:, pl.ds(0, IR), :],
                                  sem_i[par]).wait()
            pltpu.make_async_copy(img.at[pl.ds(0, 3), pl.ds(0, 1), :],
                                  iv.at[:, pl.ds(IR, 1), :],
                                  sem_i[par]).wait()

        def scatter_chunk(ch, par):
            iv, buf = bufs[par]

            def group(i2, c2):
                for u in range(2):
                    i = i2 * 2 + u
                    rl = i // (W // 16)
                    jb = (i % (W // 16)) * 16
                    px = jb + lane
                    blk = rl * W2 + (px >> 1)
                    col = (px & 1) << 2
                    hi_ok = blk >= 1
                    for cc in range(3):
                        v = iv[cc, rl, pl.ds(jb, 16)]
                        plsc.store_scatter(buf, [blk, col + cc], v)
                        plsc.store_scatter(buf, [blk - 1, col + cc + 8], v,
                                           mask=hi_ok)
                return c2

            lax.fori_loop(0, IPIX // 32, group, 0)

            lo2 = lane < 2
            last = jnp.full((16,), IROWS - 1, jnp.int32)
            colx = ((lane & 1) << 2) + 8
            for cc in range(3):
                plsc.store_scatter(buf, [last, colx + cc],
                                   iv[cc, IR, pl.ds(0, 16)], mask=lo2)

            r0 = half * 256 + ch * IR
            pltpu.async_copy(buf, table.at[pl.ds((b * H + r0) * W2, IROWS)],
                             sem_t)

        def table_drain(n):
            for _ in range(n):
                pltpu.make_async_copy(
                    bufs[0][1], table.at[pl.ds(0, IROWS)], sem_t).wait()

        img_async(0, 0)

        def body(g, carry):
            img_async(2 * g + 1, 1)
            img_wait(0)
            scatter_chunk(2 * g, 0)
            img_async(2 * g + 2, 0)
            img_wait(1)
            scatter_chunk(2 * g + 1, 1)
            table_drain(2)
            return carry

        lax.fori_loop(0, ICHUNK // 2, body, 0)
        img_wait(0)

    return interleave


def _make_warp():
    buf_set = [
        pltpu.VMEM((2, P), jnp.float32),
        pltpu.VMEM((P,), jnp.float32),
        pltpu.VMEM((P,), jnp.float32),
        pltpu.VMEM((P,), jnp.int32),
        pltpu.VMEM((NIDX,), jnp.int32),
        pltpu.VMEM((NIDX, 16), jnp.float32),
        pltpu.VMEM((3, P), jnp.float32),
    ]

    @functools.partial(
        pl.kernel,
        mesh=_MESH,
        out_type=jax.ShapeDtypeStruct((B * C, HW), jnp.float32),
        compiler_params=_CP,
        scratch_types=buf_set + buf_set + [
            pltpu.SemaphoreType.DMA,
            pltpu.SemaphoreType.DMA,
            pltpu.SemaphoreType.DMA,
            pltpu.SemaphoreType.DMA,
            pltpu.SemaphoreType.DMA,
        ],
    )
    def warp(table, flow, out, *rest):
        bufs = (rest[0:7], rest[7:14])
        sem_g = (rest[14], rest[15])
        sem_o = rest[16]
        sem_f = (rest[17], rest[18])
        wid = lax.axis_index("s") * 2 + lax.axis_index("c")
        b = wid // 2
        half = wid % 2
        lane = lax.iota(jnp.int32, 16)

        def flow_async(ch, par):
            f_v = bufs[par][0]
            off = half * PIX_PER_TILE + lax.rem(ch, NCHUNK) * P
            pltpu.async_copy(flow.at[pl.ds(2 * b, 2), pl.ds(off, P)], f_v,
                             sem_f[par])

        def flow_wait(par):
            pltpu.make_async_copy(flow.at[pl.ds(0, 2), pl.ds(0, P)],
                                  bufs[par][0], sem_f[par]).wait()

        def pass_a(ch, par):
            f_v, wx_v, wy_v, ex_v, idx_v = bufs[par][:5]
            off = half * PIX_PER_TILE + lax.rem(ch, NCHUNK) * P
            row0 = off // W
            flow_wait(par)

            def group_a(i2, c2):
                for u in range(2):
                    i = i2 * 2 + u
                    r = row0 + i // (W // 16)
                    jb = (i % (W // 16)) * 16
                    jf = (jb + lane).astype(jnp.float32)
                    fx = f_v[0, pl.ds(i * 16, 16)]
                    fy = f_v[1, pl.ds(i * 16, 16)]
                    x = jnp.clip(jf - fx, 0.0, float(W - 1))
                    y = jnp.clip(r.astype(jnp.float32) - fy,
                                 0.0, float(H - 1))
                    x0 = jnp.minimum(x.astype(jnp.int32), W - 2)
                    y0 = jnp.minimum(y.astype(jnp.int32), H - 2)
                    wx_v[pl.ds(i * 16, 16)] = x - x0.astype(jnp.float32)
                    wy_v[pl.ds(i * 16, 16)] = y - y0.astype(jnp.float32)
                    ex_v[pl.ds(i * 16, 16)] = (x0 & 1) << 2
                    rt = (b * H + y0) * W2 + (x0 >> 1)
                    pos = 2 * (i * 16 + lane)
                    plsc.store_scatter(idx_v, [pos], rt)
                    plsc.store_scatter(idx_v, [pos + 1], rt + W2)
                return c2

            lax.fori_loop(0, NG // 2, group_a, 0)

        def fire_gathers(par):
            idx_v, g_v = bufs[par][4], bufs[par][5]
            sem = sem_g[par]

            def dma_body(k, c2):
                for j in range(8):
                    d = k * 8 + j
                    pltpu.async_copy(
                        table.at[idx_v.at[pl.ds(d * 128, 128)]],
                        g_v.at[pl.ds(d * 128, 128)],
                        sem)
                return c2

            lax.fori_loop(0, NDMA // 8, dma_body, 0)

        def drain_gathers(par):
            g_v = bufs[par][5]
            pltpu.make_async_copy(table.at[pl.ds(0, NIDX)], g_v,
                                  sem_g[par]).wait()

        def pass_b(ch, par):
            wx_v, wy_v, ex_v, _, g_v, o_v = bufs[par][1:7]
            off = half * PIX_PER_TILE + ch * P

            def group_b(i2, c2):
                for u in range(2):
                    i = i2 * 2 + u
                    wx = wx_v[pl.ds(i * 16, 16)]
                    wy = wy_v[pl.ds(i * 16, 16)]
                    e = ex_v[pl.ds(i * 16, 16)]
                    wxm = 1.0 - wx
                    wym = 1.0 - wy
                    w00 = wxm * wym
                    w01 = wx * wym
                    w10 = wxm * wy
                    w11 = wx * wy
                    p2 = 2 * (i * 16 + lane)
                    for c in range(3):
                        t0 = e + c
                        t4 = t0 + 4
                        v00 = plsc.load_gather(g_v, [p2, t0])
                        v01 = plsc.load_gather(g_v, [p2, t4])
                        v10 = plsc.load_gather(g_v, [p2 + 1, t0])
                        v11 = plsc.load_gather(g_v, [p2 + 1, t4])
                        o_v[c, pl.ds(i * 16, 16)] = (
                            v00 * w00 + v01 * w01 + v10 * w10 + v11 * w11)
                return c2

            lax.fori_loop(0, NG // 2, group_b, 0)
            pltpu.async_copy(o_v, out.at[pl.ds(3 * b, 3), pl.ds(off, P)],
                             sem_o)

        def drain_outs(n):
            for _ in range(n):
                pltpu.make_async_copy(
                    bufs[0][6], out.at[pl.ds(0, 3), pl.ds(0, P)],
                    sem_o).wait()

        flow_async(0, 0)
        pass_a(0, 0)
        fire_gathers(0)
        flow_async(1, 1)

        def body(g, carry):
            pass_a(2 * g + 1, 1)
            fire_gathers(1)
            flow_async(2 * g + 2, 0)
            drain_gathers(0)
            pass_b(2 * g, 0)
            pass_a(2 * g + 2, 0)
            fire_gathers(0)
            flow_async(2 * g + 3, 1)
            drain_gathers(1)
            pass_b(2 * g + 1, 1)
            drain_outs(2)
            return carry

        lax.fori_loop(0, NCHUNK // 2, body, 0)
        drain_gathers(0)
        flow_wait(1)

    return warp


_interleave = _make_interleave()
_warp = _make_warp()


@jax.jit
def kernel(image, flow):
    img3 = image.reshape(B * C, H, W)
    table = _interleave(img3)
    flow2 = flow.reshape(B * 2, HW)
    out = _warp(table, flow2)
    return out.reshape(B, C, H, W)


# speedup vs baseline: 1.2012x; 1.2012x over previous
import functools

import jax
import jax.numpy as jnp
from jax import lax
from jax.experimental import pallas as pl
from jax.experimental.pallas import tpu as pltpu
from jax.experimental.pallas import tpu_sc as plsc

B, C, H, W = 16, 3, 512, 512
HW = H * W
W2 = W // 2
NW = 32
PIX_PER_TILE = B * HW // NW
TROWS = B * HW // 2 + 8

P = 1024
NCHUNK = PIX_PER_TILE // P
NG = P // 16
NIDX = 2 * P
NDMA = NIDX // 128

IR = 8
IPIX = IR * W
IROWS = IPIX // 2
ICHUNK = 256 // IR

_MESH = plsc.VectorSubcoreMesh(core_axis_name="c", subcore_axis_name="s")
_CP = pltpu.CompilerParams(
    needs_layout_passes=False, use_tc_tiling_on_sc=False)


def _make_interleave():
    ibuf_set = [
        pltpu.VMEM((IR + 1, W), jnp.float32),
        pltpu.VMEM((IR + 1, W), jnp.float32),
        pltpu.VMEM((IR + 1, W), jnp.float32),
        pltpu.VMEM((IROWS, 16), jnp.float32),
    ]

    @functools.partial(
        pl.kernel,
        mesh=_MESH,
        out_type=jax.ShapeDtypeStruct((TROWS, 16), jnp.float32),
        compiler_params=_CP,
        scratch_types=ibuf_set + ibuf_set + [
            pltpu.SemaphoreType.DMA,
            pltpu.SemaphoreType.DMA,
            pltpu.SemaphoreType.DMA,
        ],
    )
    def interleave(img, table, *rest):
        bufs = (rest[0:4], rest[4:8])
        sem_i = (rest[8], rest[9])
        sem_t = rest[10]
        wid = lax.axis_index("s") * 2 + lax.axis_index("c")
        b = wid // 2
        half = wid % 2

        def img_async(ch, par):
            i0, i1, i2, _ = bufs[par]
            r0 = half * 256 + lax.rem(ch, ICHUNK) * IR
            re = jnp.minimum(r0 + IR, H - 1)
            for cc, iv in ((0, i0), (1, i1), (2, i2)):
                pltpu.async_copy(img.at[3 * b + cc, pl.ds(r0, IR), :],
                                 iv.at[pl.ds(0, IR)], sem_i[par])
                pltpu.async_copy(img.at[3 * b + cc, pl.ds(re, 1), :],
                                 iv.at[pl.ds(IR, 1)], sem_i[par])

        def img_wait(par):
            i0, i1, i2, _ = bufs[par]
            for iv in (i0, i1, i2):
                pltpu.make_async_copy(img.at[0, pl.ds(0, IR), :],
                                      iv.at[pl.ds(0, IR)], sem_i[par]).wait()
                pltpu.make_async_copy(img.at[0, pl.ds(0, 1), :],
                                      iv.at[pl.ds(IR, 1)], sem_i[par]).wait()

        def scatter_chunk(ch, par):
            i0, i1, i2, buf = bufs[par]

            def group(i, c2):
                rl = i // (W // 16)
                jb = (i % (W // 16)) * 16
                lane = lax.iota(jnp.int32, 16)
                px = jb + lane
                blk = rl * W2 + (px >> 1)
                col = (px & 1) << 2
                hi_ok = blk >= 1
                for cc, iv in ((0, i0), (1, i1), (2, i2)):
                    v = iv[rl, pl.ds(jb, 16)]
                    plsc.store_scatter(buf, [blk, col + cc], v)
                    plsc.store_scatter(buf, [blk - 1, col + cc + 8], v,
                                       mask=hi_ok)
                return c2

            lax.fori_loop(0, IPIX // 16, group, 0)

            lane = lax.iota(jnp.int32, 16)
            lo2 = lane < 2
            last = jnp.full((16,), IROWS - 1, jnp.int32)
            colx = ((lane & 1) << 2) + 8
            for cc, iv in ((0, i0), (1, i1), (2, i2)):
                plsc.store_scatter(buf, [last, colx + cc],
                                   iv[IR, pl.ds(0, 16)], mask=lo2)

            r0 = half * 256 + ch * IR
            pltpu.async_copy(buf, table.at[pl.ds((b * H + r0) * W2, IROWS)],
                             sem_t)

        def table_drain(n):
            for _ in range(n):
                pltpu.make_async_copy(
                    bufs[0][3], table.at[pl.ds(0, IROWS)], sem_t).wait()

        img_async(0, 0)

        def body(g, carry):
            img_async(2 * g + 1, 1)
            img_wait(0)
            scatter_chunk(2 * g, 0)
            img_async(2 * g + 2, 0)
            img_wait(1)
            scatter_chunk(2 * g + 1, 1)
            table_drain(2)
            return carry

        lax.fori_loop(0, ICHUNK // 2, body, 0)
        img_wait(0)

    return interleave


def _make_warp():
    buf_set = [
        pltpu.VMEM((P,), jnp.float32),
        pltpu.VMEM((P,), jnp.float32),
        pltpu.VMEM((P,), jnp.float32),
        pltpu.VMEM((P,), jnp.float32),
        pltpu.VMEM((P,), jnp.int32),
        pltpu.VMEM((NIDX,), jnp.int32),
        pltpu.VMEM((NIDX, 16), jnp.float32),
        pltpu.VMEM((P,), jnp.float32),
        pltpu.VMEM((P,), jnp.float32),
        pltpu.VMEM((P,), jnp.float32),
    ]

    @functools.partial(
        pl.kernel,
        mesh=_MESH,
        out_type=jax.ShapeDtypeStruct((B * C, HW), jnp.float32),
        compiler_params=_CP,
        scratch_types=buf_set + buf_set + [
            pltpu.SemaphoreType.DMA,
            pltpu.SemaphoreType.DMA,
            pltpu.SemaphoreType.DMA,
            pltpu.SemaphoreType.DMA,
            pltpu.SemaphoreType.DMA,
        ],
    )
    def warp(table, flow, out, *rest):
        bufs = (rest[0:10], rest[10:20])
        sem_g = (rest[20], rest[21])
        sem_o = rest[22]
        sem_f = (rest[23], rest[24])
        wid = lax.axis_index("s") * 2 + lax.axis_index("c")
        b = wid // 2
        half = wid % 2

        def flow_async(ch, par):
            fx_v, fy_v = bufs[par][0], bufs[par][1]
            off = half * PIX_PER_TILE + lax.rem(ch, NCHUNK) * P
            pltpu.async_copy(flow.at[2 * b, pl.ds(off, P)], fx_v, sem_f[par])
            pltpu.async_copy(flow.at[2 * b + 1, pl.ds(off, P)], fy_v,
                             sem_f[par])

        def flow_wait(par):
            fx_v, fy_v = bufs[par][0], bufs[par][1]
            pltpu.make_async_copy(flow.at[0, pl.ds(0, P)], fx_v,
                                  sem_f[par]).wait()
            pltpu.make_async_copy(flow.at[0, pl.ds(0, P)], fy_v,
                                  sem_f[par]).wait()

        def pass_a(ch, par):
            fx_v, fy_v, wx_v, wy_v, ex_v, idx_v = bufs[par][:6]
            off = half * PIX_PER_TILE + lax.rem(ch, NCHUNK) * P
            row0 = off // W
            flow_wait(par)

            def group_a(i, c2):
                r = row0 + i // (W // 16)
                jb = (i % (W // 16)) * 16
                lane = lax.iota(jnp.int32, 16)
                jf = (jb + lane).astype(jnp.float32)
                fx = fx_v[pl.ds(i * 16, 16)]
                fy = fy_v[pl.ds(i * 16, 16)]
                x = jnp.clip(jf - fx, 0.0, float(W - 1))
                y = jnp.clip(r.astype(jnp.float32) - fy, 0.0, float(H - 1))
                x0 = jnp.minimum(x.astype(jnp.int32), W - 2)
                y0 = jnp.minimum(y.astype(jnp.int32), H - 2)
                wx_v[pl.ds(i * 16, 16)] = x - x0.astype(jnp.float32)
                wy_v[pl.ds(i * 16, 16)] = y - y0.astype(jnp.float32)
                ex_v[pl.ds(i * 16, 16)] = (x0 & 1) << 2
                rt = (b * H + y0) * W2 + (x0 >> 1)
                pos = 2 * (i * 16 + lane)
                plsc.store_scatter(idx_v, [pos], rt)
                plsc.store_scatter(idx_v, [pos + 1], rt + W2)
                return c2

            lax.fori_loop(0, NG, group_a, 0)

        def fire_gathers(par):
            idx_v, g_v = bufs[par][5], bufs[par][6]
            sem = sem_g[par]

            def dma_body(k, c2):
                for j in range(8):
                    d = k * 8 + j
                    pltpu.async_copy(
                        table.at[idx_v.at[pl.ds(d * 128, 128)]],
                        g_v.at[pl.ds(d * 128, 128)],
                        sem)
                return c2

            lax.fori_loop(0, NDMA // 8, dma_body, 0)

        def drain_gathers(par):
            g_v = bufs[par][6]
            pltpu.make_async_copy(table.at[pl.ds(0, NIDX)], g_v,
                                  sem_g[par]).wait()

        def pass_b(ch, par):
            wx_v, wy_v, ex_v, _, g_v, o0, o1, o2 = bufs[par][2:10]
            off = half * PIX_PER_TILE + ch * P

            def group_b(i, c2):
                lane = lax.iota(jnp.int32, 16)
                wx = wx_v[pl.ds(i * 16, 16)]
                wy = wy_v[pl.ds(i * 16, 16)]
                e = ex_v[pl.ds(i * 16, 16)]
                wxm = 1.0 - wx
                wym = 1.0 - wy
                w00 = wxm * wym
                w01 = wx * wym
                w10 = wxm * wy
                w11 = wx * wy
                p2 = 2 * (i * 16 + lane)
                res = []
                for c in range(3):
                    t0 = e + c
                    t4 = t0 + 4
                    v00 = plsc.load_gather(g_v, [p2, t0])
                    v01 = plsc.load_gather(g_v, [p2, t4])
                    v10 = plsc.load_gather(g_v, [p2 + 1, t0])
                    v11 = plsc.load_gather(g_v, [p2 + 1, t4])
                    res.append(v00 * w00 + v01 * w01 + v10 * w10 + v11 * w11)
                o0[pl.ds(i * 16, 16)] = res[0]
                o1[pl.ds(i * 16, 16)] = res[1]
                o2[pl.ds(i * 16, 16)] = res[2]
                return c2

            lax.fori_loop(0, NG, group_b, 0)
            pltpu.async_copy(o0, out.at[3 * b, pl.ds(off, P)], sem_o)
            pltpu.async_copy(o1, out.at[3 * b + 1, pl.ds(off, P)], sem_o)
            pltpu.async_copy(o2, out.at[3 * b + 2, pl.ds(off, P)], sem_o)

        def drain_outs(n):
            for _ in range(n):
                pltpu.make_async_copy(
                    bufs[0][7], out.at[3 * b, pl.ds(0, P)], sem_o).wait()

        flow_async(0, 0)
        pass_a(0, 0)
        fire_gathers(0)
        flow_async(1, 1)

        def body(g, carry):
            pass_a(2 * g + 1, 1)
            fire_gathers(1)
            flow_async(2 * g + 2, 0)
            drain_gathers(0)
            pass_b(2 * g, 0)
            pass_a(2 * g + 2, 0)
            fire_gathers(0)
            flow_async(2 * g + 3, 1)
            drain_gathers(1)
            pass_b(2 * g + 1, 1)
            drain_outs(6)
            return carry

        lax.fori_loop(0, NCHUNK // 2, body, 0)
        drain_gathers(0)
        flow_wait(1)

    return warp


_interleave = _make_interleave()
_warp = _make_warp()


@jax.jit
def kernel(image, flow):
    img3 = image.reshape(B * C, H, W)
    table = _interleave(img3)
    flow2 = flow.reshape(B * 2, HW)
    out = _warp(table, flow2)
    return out.reshape(B, C, H, W)


# speedup vs baseline: 1.2576x; 1.0470x over previous
import functools

import jax
import jax.numpy as jnp
from jax import lax
from jax.experimental import pallas as pl
from jax.experimental.pallas import tpu as pltpu
from jax.experimental.pallas import tpu_sc as plsc

B, C, H, W = 16, 3, 512, 512
HW = H * W
W2 = W // 2
NW = 32
PIX_PER_TILE = B * HW // NW
TROWS = B * HW // 2 + 8

P = 1024
NCHUNK = PIX_PER_TILE // P
NG = P // 16
NIDX = 2 * P
NDMA = NIDX // 128

IR = 8
IPIX = IR * W
IROWS = IPIX // 2
ICHUNK = 256 // IR

_MESH = plsc.VectorSubcoreMesh(core_axis_name="c", subcore_axis_name="s")
_CP = pltpu.CompilerParams(
    needs_layout_passes=False, use_tc_tiling_on_sc=False)


def _make_interleave():
    ibuf_set = [
        pltpu.VMEM((IR + 1, W), jnp.float32),
        pltpu.VMEM((IR + 1, W), jnp.float32),
        pltpu.VMEM((IR + 1, W), jnp.float32),
        pltpu.VMEM((IROWS, 16), jnp.float32),
    ]

    @functools.partial(
        pl.kernel,
        mesh=_MESH,
        out_type=jax.ShapeDtypeStruct((TROWS, 16), jnp.float32),
        compiler_params=_CP,
        scratch_types=ibuf_set + ibuf_set + [
            pltpu.SemaphoreType.DMA,
            pltpu.SemaphoreType.DMA,
            pltpu.SemaphoreType.DMA,
        ],
    )
    def interleave(img, table, *rest):
        bufs = (rest[0:4], rest[4:8])
        sem_i = (rest[8], rest[9])
        sem_t = rest[10]
        wid = lax.axis_index("s") * 2 + lax.axis_index("c")
        b = wid // 2
        half = wid % 2

        def img_async(ch, par):
            i0, i1, i2, _ = bufs[par]
            r0 = half * 256 + lax.rem(ch, ICHUNK) * IR
            re = jnp.minimum(r0 + IR, H - 1)
            for cc, iv in ((0, i0), (1, i1), (2, i2)):
                pltpu.async_copy(img.at[3 * b + cc, pl.ds(r0, IR), :],
                                 iv.at[pl.ds(0, IR)], sem_i[par])
                pltpu.async_copy(img.at[3 * b + cc, pl.ds(re, 1), :],
                                 iv.at[pl.ds(IR, 1)], sem_i[par])

        def img_wait(par):
            i0, i1, i2, _ = bufs[par]
            for iv in (i0, i1, i2):
                pltpu.make_async_copy(img.at[0, pl.ds(0, IR), :],
                                      iv.at[pl.ds(0, IR)], sem_i[par]).wait()
                pltpu.make_async_copy(img.at[0, pl.ds(0, 1), :],
                                      iv.at[pl.ds(IR, 1)], sem_i[par]).wait()

        def scatter_chunk(ch, par):
            i0, i1, i2, buf = bufs[par]

            def group(i, c2):
                rl = i // (W // 16)
                jb = (i % (W // 16)) * 16
                lane = lax.iota(jnp.int32, 16)
                px = jb + lane
                blk = rl * W2 + (px >> 1)
                col = (px & 1) << 2
                hi_ok = blk >= 1
                for cc, iv in ((0, i0), (1, i1), (2, i2)):
                    v = iv[rl, pl.ds(jb, 16)]
                    plsc.store_scatter(buf, [blk, col + cc], v)
                    plsc.store_scatter(buf, [blk - 1, col + cc + 8], v,
                                       mask=hi_ok)
                return c2

            lax.fori_loop(0, IPIX // 16, group, 0)

            lane = lax.iota(jnp.int32, 16)
            lo2 = lane < 2
            last = jnp.full((16,), IROWS - 1, jnp.int32)
            colx = ((lane & 1) << 2) + 8
            for cc, iv in ((0, i0), (1, i1), (2, i2)):
                plsc.store_scatter(buf, [last, colx + cc],
                                   iv[IR, pl.ds(0, 16)], mask=lo2)

            r0 = half * 256 + ch * IR
            pltpu.async_copy(buf, table.at[pl.ds((b * H + r0) * W2, IROWS)],
                             sem_t)

        def table_drain(n):
            for _ in range(n):
                pltpu.make_async_copy(
                    bufs[0][3], table.at[pl.ds(0, IROWS)], sem_t).wait()

        img_async(0, 0)

        def body(g, carry):
            img_async(2 * g + 1, 1)
            img_wait(0)
            scatter_chunk(2 * g, 0)
            img_async(2 * g + 2, 0)
            img_wait(1)
            scatter_chunk(2 * g + 1, 1)
            table_drain(2)
            return carry

        lax.fori_loop(0, ICHUNK // 2, body, 0)
        img_wait(0)

    return interleave


def _make_warp():
    buf_set = [
        pltpu.VMEM((P,), jnp.float32),
        pltpu.VMEM((P,), jnp.float32),
        pltpu.VMEM((P,), jnp.float32),
        pltpu.VMEM((P,), jnp.float32),
        pltpu.VMEM((P,), jnp.int32),
        pltpu.VMEM((NIDX,), jnp.int32),
        pltpu.VMEM((NIDX, 16), jnp.float32),
        pltpu.VMEM((P,), jnp.float32),
        pltpu.VMEM((P,), jnp.float32),
        pltpu.VMEM((P,), jnp.float32),
    ]

    @functools.partial(
        pl.kernel,
        mesh=_MESH,
        out_type=jax.ShapeDtypeStruct((B * C, HW), jnp.float32),
        compiler_params=_CP,
        scratch_types=buf_set + buf_set + [
            pltpu.SemaphoreType.DMA,
            pltpu.SemaphoreType.DMA,
            pltpu.SemaphoreType.DMA,
            pltpu.SemaphoreType.DMA,
            pltpu.SemaphoreType.DMA,
        ],
    )
    def warp(table, flow, out, *rest):
        bufs = (rest[0:10], rest[10:20])
        sem_g = (rest[20], rest[21])
        sem_o = rest[22]
        sem_f = (rest[23], rest[24])
        wid = lax.axis_index("s") * 2 + lax.axis_index("c")
        b = wid // 2
        half = wid % 2

        def flow_async(ch, par):
            fx_v, fy_v = bufs[par][0], bufs[par][1]
            off = half * PIX_PER_TILE + lax.rem(ch, NCHUNK) * P
            pltpu.async_copy(flow.at[2 * b, pl.ds(off, P)], fx_v, sem_f[par])
            pltpu.async_copy(flow.at[2 * b + 1, pl.ds(off, P)], fy_v,
                             sem_f[par])

        def flow_wait(par):
            fx_v, fy_v = bufs[par][0], bufs[par][1]
            pltpu.make_async_copy(flow.at[0, pl.ds(0, P)], fx_v,
                                  sem_f[par]).wait()
            pltpu.make_async_copy(flow.at[0, pl.ds(0, P)], fy_v,
                                  sem_f[par]).wait()

        def pass_a(ch, par):
            fx_v, fy_v, wx_v, wy_v, ex_v, idx_v = bufs[par][:6]
            off = half * PIX_PER_TILE + lax.rem(ch, NCHUNK) * P
            row0 = off // W
            flow_wait(par)

            def group_a(i, c2):
                r = row0 + i // (W // 16)
                jb = (i % (W // 16)) * 16
                lane = lax.iota(jnp.int32, 16)
                jf = (jb + lane).astype(jnp.float32)
                fx = fx_v[pl.ds(i * 16, 16)]
                fy = fy_v[pl.ds(i * 16, 16)]
                x = jnp.clip(jf - fx, 0.0, float(W - 1))
                y = jnp.clip(r.astype(jnp.float32) - fy, 0.0, float(H - 1))
                x0 = jnp.minimum(x.astype(jnp.int32), W - 2)
                y0 = jnp.minimum(y.astype(jnp.int32), H - 2)
                wx_v[pl.ds(i * 16, 16)] = x - x0.astype(jnp.float32)
                wy_v[pl.ds(i * 16, 16)] = y - y0.astype(jnp.float32)
                ex_v[pl.ds(i * 16, 16)] = (x0 & 1) << 2
                rt = (b * H + y0) * W2 + (x0 >> 1)
                pos = 2 * (i * 16 + lane)
                plsc.store_scatter(idx_v, [pos], rt)
                plsc.store_scatter(idx_v, [pos + 1], rt + W2)
                return c2

            lax.fori_loop(0, NG, group_a, 0)

        def fire_gathers(par):
            idx_v, g_v = bufs[par][5], bufs[par][6]
            sem = sem_g[par]

            def dma_body(k, c2):
                for j in range(8):
                    d = k * 8 + j
                    pltpu.async_copy(
                        table.at[idx_v.at[pl.ds(d * 128, 128)]],
                        g_v.at[pl.ds(d * 128, 128)],
                        sem)
                return c2

            lax.fori_loop(0, NDMA // 8, dma_body, 0)

        def drain_gathers(par):
            g_v = bufs[par][6]
            pltpu.make_async_copy(table.at[pl.ds(0, NIDX)], g_v,
                                  sem_g[par]).wait()

        def pass_b(ch, par):
            wx_v, wy_v, ex_v, _, g_v, o0, o1, o2 = bufs[par][2:10]
            off = half * PIX_PER_TILE + ch * P

            def group_b(i2, c2):
                lane = lax.iota(jnp.int32, 16)
                for u in range(2):
                    i = i2 * 2 + u
                    wx = wx_v[pl.ds(i * 16, 16)]
                    wy = wy_v[pl.ds(i * 16, 16)]
                    e = ex_v[pl.ds(i * 16, 16)]
                    wxm = 1.0 - wx
                    wym = 1.0 - wy
                    w00 = wxm * wym
                    w01 = wx * wym
                    w10 = wxm * wy
                    w11 = wx * wy
                    p2 = 2 * (i * 16 + lane)
                    res = []
                    for c in range(3):
                        t0 = e + c
                        t4 = t0 + 4
                        v00 = plsc.load_gather(g_v, [p2, t0])
                        v01 = plsc.load_gather(g_v, [p2, t4])
                        v10 = plsc.load_gather(g_v, [p2 + 1, t0])
                        v11 = plsc.load_gather(g_v, [p2 + 1, t4])
                        res.append(
                            v00 * w00 + v01 * w01 + v10 * w10 + v11 * w11)
                    o0[pl.ds(i * 16, 16)] = res[0]
                    o1[pl.ds(i * 16, 16)] = res[1]
                    o2[pl.ds(i * 16, 16)] = res[2]
                return c2

            lax.fori_loop(0, NG // 2, group_b, 0)
            pltpu.async_copy(o0, out.at[3 * b, pl.ds(off, P)], sem_o)
            pltpu.async_copy(o1, out.at[3 * b + 1, pl.ds(off, P)], sem_o)
            pltpu.async_copy(o2, out.at[3 * b + 2, pl.ds(off, P)], sem_o)

        def drain_outs(n):
            for _ in range(n):
                pltpu.make_async_copy(
                    bufs[0][7], out.at[3 * b, pl.ds(0, P)], sem_o).wait()

        flow_async(0, 0)
        pass_a(0, 0)
        fire_gathers(0)
        flow_async(1, 1)

        def body(g, carry):
            pass_a(2 * g + 1, 1)
            fire_gathers(1)
            flow_async(2 * g + 2, 0)
            drain_gathers(0)
            pass_b(2 * g, 0)
            pass_a(2 * g + 2, 0)
            fire_gathers(0)
            flow_async(2 * g + 3, 1)
            drain_gathers(1)
            pass_b(2 * g + 1, 1)
            drain_outs(6)
            return carry

        lax.fori_loop(0, NCHUNK // 2, body, 0)
        drain_gathers(0)
        flow_wait(1)

    return warp


_interleave = _make_interleave()
_warp = _make_warp()


@jax.jit
def kernel(image, flow):
    img3 = image.reshape(B * C, H, W)
    table = _interleave(img3)
    flow2 = flow.reshape(B * 2, HW)
    out = _warp(table, flow2)
    return out.reshape(B, C, H, W)
